# Initial kernel scaffold; baseline (speedup 1.0000x reference)
import jax
import jax.numpy as jnp
from jax.experimental import pallas as pl


def kernel(x, edge_index, edge_attr, W0, b0, W1, b1, W2, b2):
    raise NotImplementedError("write your pallas kernel here")


# speedup vs baseline: 2.9195x; 2.9195x over previous
import functools

import jax
import jax.numpy as jnp
from jax import lax
from jax.experimental import pallas as pl
from jax.experimental.pallas import tpu as pltpu
from jax.experimental.pallas import tpu_sc as plsc

N = 10000
E = 320000
DF = 128
DE = 16

NC = 2
NS = 16
NW = NC * NS

B = 128
NCH = 80
EPW = NCH * B
E_PAD = NW * EPW

ROWS_PER_TILE = 632
N_PAD = NS * ROWS_PER_TILE

EA_W = 128


def _sc_agg_body(y_hbm, src_hbm, dst_hbm, z128_hbm, g_out,
                 src_v, dst_v, msg_v, acc, gsem):
    cid = lax.axis_index("c")
    sid = lax.axis_index("s")
    wid = cid * NS + sid
    rows = pl.ds(sid * ROWS_PER_TILE, ROWS_PER_TILE)

    pltpu.sync_copy(z128_hbm.at[rows], acc.at[rows])
    pltpu.sync_copy(src_hbm.at[wid], src_v)
    pltpu.sync_copy(dst_hbm.at[wid], dst_v)
    plsc.subcore_barrier()

    def chunk(j, carry):
        pltpu.async_copy(y_hbm.at[src_v.at[j]], msg_v, gsem).wait()
        pltpu.sync_copy(msg_v, acc.at[dst_v.at[j]], add=True)
        return carry

    lax.fori_loop(0, NCH, chunk, 0)
    plsc.subcore_barrier()
    pltpu.sync_copy(acc.at[rows], g_out.at[cid, rows])


def _sc_ea_body(ea_hbm, dst_hbm, z24_hbm, ea_out,
                dst_v, ea_v, acc2):
    cid = lax.axis_index("c")
    sid = lax.axis_index("s")
    wid = cid * NS + sid
    rows = pl.ds(sid * ROWS_PER_TILE, ROWS_PER_TILE)

    pltpu.sync_copy(z24_hbm.at[rows], acc2.at[rows])
    pltpu.sync_copy(dst_hbm.at[wid], dst_v)
    plsc.subcore_barrier()

    def chunk(j, carry):
        pltpu.sync_copy(ea_hbm.at[wid * NCH + j], ea_v)
        pltpu.sync_copy(ea_v, acc2.at[dst_v.at[j]], add=True)
        return carry

    lax.fori_loop(0, NCH, chunk, 0)
    plsc.subcore_barrier()
    pltpu.sync_copy(acc2.at[rows], ea_out.at[cid, rows])


_mesh = plsc.VectorSubcoreMesh(core_axis_name="c", subcore_axis_name="s")

_sc_agg = pl.kernel(
    _sc_agg_body,
    out_type=jax.ShapeDtypeStruct((NC, N_PAD, DF), jnp.float32),
    mesh=_mesh,
    scratch_types=[
        pltpu.VMEM((NCH, B), jnp.int32),
        pltpu.VMEM((NCH, B), jnp.int32),
        pltpu.VMEM((B, DF), jnp.float32),
        pltpu.VMEM_SHARED((N_PAD, DF), jnp.float32),
        pltpu.SemaphoreType.DMA,
    ],
    name="sc_gcn_agg",
)

_sc_ea = pl.kernel(
    _sc_ea_body,
    out_type=jax.ShapeDtypeStruct((NC, N_PAD, EA_W), jnp.float32),
    mesh=_mesh,
    scratch_types=[
        pltpu.VMEM((NCH, B), jnp.int32),
        pltpu.VMEM((B, EA_W), jnp.float32),
        pltpu.VMEM_SHARED((N_PAD, EA_W), jnp.float32),
    ],
    name="sc_gcn_ea",
)


R = 1000


def _mm_body(x_ref, w_ref, y_ref):
    y_ref[...] = jnp.dot(x_ref[...], w_ref[...],
                         preferred_element_type=jnp.float32)


_mm = pl.pallas_call(
    _mm_body,
    grid=(N // R,),
    in_specs=[pl.BlockSpec((R, DF), lambda i: (i, 0)),
              pl.BlockSpec((DF, DF), lambda i: (0, 0))],
    out_specs=pl.BlockSpec((R, DF), lambda i: (i, 0)),
    out_shape=jax.ShapeDtypeStruct((N, DF), jnp.float32),
)


def _stage_body(residual, emit_y, *refs):
    it = iter(refs)
    g0 = next(it)[...]
    g1 = next(it)[...]
    ea0 = next(it)[...]
    ea1 = next(it)[...]
    xp = next(it)[...] if residual else None
    we = next(it)[...]
    b = next(it)[...]
    wx = next(it)[...] if emit_y else None
    out_refs = list(it)

    g = g0 + g1
    ea = ea0 + ea1
    s = ea[:, :DE]
    deg = ea[:, DE:DE + 1]
    invd = 1.0 / jnp.maximum(deg, 1.0)
    m = (g + jnp.dot(s, we, preferred_element_type=jnp.float32)
         + deg * b) * invd
    if residual:
        xnew = xp + jnp.maximum(m, 0.0)
    else:
        xnew = m
    out_refs[0][...] = xnew
    if emit_y:
        out_refs[1][...] = jnp.dot(xnew, wx,
                                   preferred_element_type=jnp.float32)


def _make_stage(residual, emit_y):
    row = lambda i: (i, 0)
    fix = lambda i: (0, 0)
    in_specs = [pl.BlockSpec((R, DF), row),
                pl.BlockSpec((R, DF), row),
                pl.BlockSpec((R, EA_W), row),
                pl.BlockSpec((R, EA_W), row)]
    if residual:
        in_specs.append(pl.BlockSpec((R, DF), row))
    in_specs += [pl.BlockSpec((DE, DF), fix),
                 pl.BlockSpec((1, DF), fix)]
    if emit_y:
        in_specs.append(pl.BlockSpec((DF, DF), fix))
    n_out = 2 if emit_y else 1
    out_specs = [pl.BlockSpec((R, DF), row)] * n_out
    out_shape = [jax.ShapeDtypeStruct((N, DF), jnp.float32)] * n_out
    if n_out == 1:
        out_specs, out_shape = out_specs[0], out_shape[0]
    return pl.pallas_call(
        functools.partial(_stage_body, residual, emit_y),
        grid=(N // R,),
        in_specs=in_specs,
        out_specs=out_specs,
        out_shape=out_shape,
    )


_stage_first = _make_stage(residual=False, emit_y=True)
_stage_mid = _make_stage(residual=True, emit_y=True)
_stage_last = _make_stage(residual=True, emit_y=False)


def kernel(x, edge_index, edge_attr, W0, b0, W1, b1, W2, b2):
    src = edge_index[0].astype(jnp.int32)
    dst = edge_index[1].astype(jnp.int32)
    pad = E_PAD - E
    src_p = jnp.concatenate([src, jnp.zeros((pad,), jnp.int32)]
                            ).reshape(NW, NCH, B)
    dst_p = jnp.concatenate([dst, jnp.full((pad,), N, jnp.int32)]
                            ).reshape(NW, NCH, B)
    ea_aug = jnp.concatenate(
        [edge_attr,
         jnp.ones((E, 1), jnp.float32),
         jnp.zeros((E, EA_W - DE - 1), jnp.float32)], axis=1)
    ea_p = jnp.concatenate([ea_aug, jnp.zeros((pad, EA_W), jnp.float32)]
                           ).reshape(NW * NCH, B, EA_W)
    z128 = jnp.zeros((N_PAD, DF), jnp.float32)
    z24 = jnp.zeros((N_PAD, EA_W), jnp.float32)

    Wx0, We0 = W0[:DF], W0[DF:]
    Wx1, We1 = W1[:DF], W1[DF:]
    Wx2, We2 = W2[:DF], W2[DF:]
    b0r = b0.reshape(1, DF)
    b1r = b1.reshape(1, DF)
    b2r = b2.reshape(1, DF)

    zrow = jnp.zeros((N, DF), jnp.float32)
    zea = jnp.zeros((N, EA_W), jnp.float32)

    def agg_jax(y):
        return jax.ops.segment_sum(jnp.take(y, src, axis=0), dst,
                                   num_segments=N)

    y0 = _mm(x, Wx0)
    eacc = _sc_ea(ea_p, dst_p, z24)
    ea0, ea1 = eacc[0, :N], eacc[1, :N]
    g = _sc_agg(y0, src_p, dst_p, z128)
    x1, y1 = _stage_first(g[0, :N], g[1, :N], ea0, ea1, We0, b0r, Wx1)
    g = _sc_agg(y1, src_p, dst_p, z128)
    x2, y2 = _stage_mid(g[0, :N], g[1, :N], ea0, ea1, x1, We1, b1r, Wx2)
    g = _sc_agg(y2, src_p, dst_p, z128)
    x3 = _stage_last(g[0, :N], g[1, :N], ea0, ea1, x2, We2, b2r)
    return x3


# speedup vs baseline: 3.2949x; 1.1286x over previous
import functools

import jax
import jax.numpy as jnp
from jax import lax
from jax.experimental import pallas as pl
from jax.experimental.pallas import tpu as pltpu
from jax.experimental.pallas import tpu_sc as plsc

N = 10000
E = 320000
DF = 128
DE = 16

NC = 2
NS = 16
NW = NC * NS

B = 128
NCH = 80
NPH = 2
NCH2 = NCH // NPH
EPW = NCH * B
E_PAD = NW * EPW

ROWS_PER_TILE = 632
N_PAD = NS * ROWS_PER_TILE

EA_W = 128


def _sc_agg_body(y_hbm, src_hbm, dst_hbm, z128_hbm, g_out,
                 src_v, dst_v, msg0, msg1, acc, sem0, sem1):
    cid = lax.axis_index("c")
    sid = lax.axis_index("s")
    wid = cid * NS + sid
    rows = pl.ds(sid * ROWS_PER_TILE, ROWS_PER_TILE)

    pltpu.sync_copy(z128_hbm.at[rows], acc.at[rows])
    plsc.subcore_barrier()

    for p in range(NPH):
        blk = wid * NPH + p
        pltpu.sync_copy(src_hbm.at[blk], src_v)
        pltpu.sync_copy(dst_hbm.at[blk], dst_v)

        pltpu.async_copy(y_hbm.at[src_v.at[0]], msg0, sem0)
        pltpu.async_copy(y_hbm.at[src_v.at[1]], msg1, sem1)

        def chunk(i, carry):
            j0 = 2 * i
            j1 = j0 + 1
            pltpu.make_async_copy(y_hbm.at[src_v.at[j0]], msg0, sem0).wait()
            pltpu.sync_copy(msg0, acc.at[dst_v.at[j0]], add=True)
            pltpu.async_copy(
                y_hbm.at[src_v.at[jnp.minimum(j0 + 2, NCH2 - 2)]], msg0, sem0)
            pltpu.make_async_copy(y_hbm.at[src_v.at[j1]], msg1, sem1).wait()
            pltpu.sync_copy(msg1, acc.at[dst_v.at[j1]], add=True)
            pltpu.async_copy(
                y_hbm.at[src_v.at[jnp.minimum(j1 + 2, NCH2 - 1)]], msg1, sem1)
            return carry

        lax.fori_loop(0, NCH2 // 2, chunk, 0)
        pltpu.make_async_copy(y_hbm.at[src_v.at[NCH2 - 2]], msg0, sem0).wait()
        pltpu.make_async_copy(y_hbm.at[src_v.at[NCH2 - 1]], msg1, sem1).wait()

    plsc.subcore_barrier()
    pltpu.sync_copy(acc.at[rows], g_out.at[cid, rows])


def _sc_ea_body(ea_hbm, dst_hbm, z128_hbm, ea_out,
                dst_v, ea0_v, ea1_v, acc2, sem0, sem1):
    cid = lax.axis_index("c")
    sid = lax.axis_index("s")
    wid = cid * NS + sid
    rows = pl.ds(sid * ROWS_PER_TILE, ROWS_PER_TILE)

    pltpu.sync_copy(z128_hbm.at[rows], acc2.at[rows])
    for p in range(NPH):
        pltpu.sync_copy(dst_hbm.at[wid * NPH + p],
                        dst_v.at[pl.ds(p * NCH2, NCH2)])
    plsc.subcore_barrier()

    base = wid * NCH
    pltpu.async_copy(ea_hbm.at[base], ea0_v, sem0)
    pltpu.async_copy(ea_hbm.at[base + 1], ea1_v, sem1)

    def chunk(i, carry):
        j0 = 2 * i
        j1 = j0 + 1
        pltpu.make_async_copy(ea_hbm.at[base + j0], ea0_v, sem0).wait()
        pltpu.sync_copy(ea0_v, acc2.at[dst_v.at[j0]], add=True)
        pltpu.async_copy(
            ea_hbm.at[base + jnp.minimum(j0 + 2, NCH - 2)], ea0_v, sem0)
        pltpu.make_async_copy(ea_hbm.at[base + j1], ea1_v, sem1).wait()
        pltpu.sync_copy(ea1_v, acc2.at[dst_v.at[j1]], add=True)
        pltpu.async_copy(
            ea_hbm.at[base + jnp.minimum(j1 + 2, NCH - 1)], ea1_v, sem1)
        return carry

    lax.fori_loop(0, NCH // 2, chunk, 0)
    pltpu.make_async_copy(ea_hbm.at[base + NCH - 2], ea0_v, sem0).wait()
    pltpu.make_async_copy(ea_hbm.at[base + NCH - 1], ea1_v, sem1).wait()
    plsc.subcore_barrier()
    pltpu.sync_copy(acc2.at[rows], ea_out.at[cid, rows])


_mesh = plsc.VectorSubcoreMesh(core_axis_name="c", subcore_axis_name="s")

_sc_agg = pl.kernel(
    _sc_agg_body,
    out_type=jax.ShapeDtypeStruct((NC, N_PAD, DF), jnp.float32),
    mesh=_mesh,
    scratch_types=[
        pltpu.VMEM((NCH2, B), jnp.int32),
        pltpu.VMEM((NCH2, B), jnp.int32),
        pltpu.VMEM((B, DF), jnp.float32),
        pltpu.VMEM((B, DF), jnp.float32),
        pltpu.VMEM_SHARED((N_PAD, DF), jnp.float32),
        pltpu.SemaphoreType.DMA,
        pltpu.SemaphoreType.DMA,
    ],
    name="sc_gcn_agg",
)

_sc_ea = pl.kernel(
    _sc_ea_body,
    out_type=jax.ShapeDtypeStruct((NC, N_PAD, EA_W), jnp.float32),
    mesh=_mesh,
    scratch_types=[
        pltpu.VMEM((NCH, B), jnp.int32),
        pltpu.VMEM((B, EA_W), jnp.float32),
        pltpu.VMEM((B, EA_W), jnp.float32),
        pltpu.VMEM_SHARED((N_PAD, EA_W), jnp.float32),
        pltpu.SemaphoreType.DMA,
        pltpu.SemaphoreType.DMA,
    ],
    name="sc_gcn_ea",
)


R = 1000


def _mm_body(x_ref, w_ref, y_ref):
    y_ref[...] = jnp.dot(x_ref[...], w_ref[...],
                         preferred_element_type=jnp.float32)


_mm = pl.pallas_call(
    _mm_body,
    grid=(N // R,),
    in_specs=[pl.BlockSpec((R, DF), lambda i: (i, 0)),
              pl.BlockSpec((DF, DF), lambda i: (0, 0))],
    out_specs=pl.BlockSpec((R, DF), lambda i: (i, 0)),
    out_shape=jax.ShapeDtypeStruct((N, DF), jnp.float32),
)


def _stage_body(residual, emit_y, *refs):
    it = iter(refs)
    g0 = next(it)[...]
    g1 = next(it)[...]
    ea0 = next(it)[...]
    ea1 = next(it)[...]
    xp = next(it)[...] if residual else None
    we = next(it)[...]
    b = next(it)[...]
    wx = next(it)[...] if emit_y else None
    out_refs = list(it)

    g = g0 + g1
    ea = ea0 + ea1
    s = ea[:, :DE]
    deg = ea[:, DE:DE + 1]
    invd = 1.0 / jnp.maximum(deg, 1.0)
    m = (g + jnp.dot(s, we, preferred_element_type=jnp.float32)
         + deg * b) * invd
    if residual:
        xnew = xp + jnp.maximum(m, 0.0)
    else:
        xnew = m
    out_refs[0][...] = xnew
    if emit_y:
        out_refs[1][...] = jnp.dot(xnew, wx,
                                   preferred_element_type=jnp.float32)


def _make_stage(residual, emit_y):
    row = lambda i: (i, 0)
    fix = lambda i: (0, 0)
    in_specs = [pl.BlockSpec((R, DF), row),
                pl.BlockSpec((R, DF), row),
                pl.BlockSpec((R, EA_W), row),
                pl.BlockSpec((R, EA_W), row)]
    if residual:
        in_specs.append(pl.BlockSpec((R, DF), row))
    in_specs += [pl.BlockSpec((DE, DF), fix),
                 pl.BlockSpec((1, DF), fix)]
    if emit_y:
        in_specs.append(pl.BlockSpec((DF, DF), fix))
    n_out = 2 if emit_y else 1
    out_specs = [pl.BlockSpec((R, DF), row)] * n_out
    out_shape = [jax.ShapeDtypeStruct((N, DF), jnp.float32)] * n_out
    if n_out == 1:
        out_specs, out_shape = out_specs[0], out_shape[0]
    return pl.pallas_call(
        functools.partial(_stage_body, residual, emit_y),
        grid=(N // R,),
        in_specs=in_specs,
        out_specs=out_specs,
        out_shape=out_shape,
    )


_stage_first = _make_stage(residual=False, emit_y=True)
_stage_mid = _make_stage(residual=True, emit_y=True)
_stage_last = _make_stage(residual=True, emit_y=False)


def kernel(x, edge_index, edge_attr, W0, b0, W1, b1, W2, b2):
    src = edge_index[0].astype(jnp.int32)
    dst = edge_index[1].astype(jnp.int32)
    pad = E_PAD - E
    src_p = jnp.concatenate([src, jnp.zeros((pad,), jnp.int32)]
                            ).reshape(NW * NPH, NCH2, B)
    dst_p = jnp.concatenate([dst, jnp.full((pad,), N, jnp.int32)]
                            ).reshape(NW * NPH, NCH2, B)
    ea_aug = jnp.concatenate(
        [edge_attr,
         jnp.ones((E, 1), jnp.float32),
         jnp.zeros((E, EA_W - DE - 1), jnp.float32)], axis=1)
    ea_p = jnp.concatenate([ea_aug, jnp.zeros((pad, EA_W), jnp.float32)]
                           ).reshape(NW * NCH, B, EA_W)
    z128 = jnp.zeros((N_PAD, DF), jnp.float32)

    Wx0, We0 = W0[:DF], W0[DF:]
    Wx1, We1 = W1[:DF], W1[DF:]
    Wx2, We2 = W2[:DF], W2[DF:]
    b0r = b0.reshape(1, DF)
    b1r = b1.reshape(1, DF)
    b2r = b2.reshape(1, DF)

    y0 = _mm(x, Wx0)
    eacc = _sc_ea(ea_p, dst_p, z128)
    ea0, ea1 = eacc[0, :N], eacc[1, :N]
    g = _sc_agg(y0, src_p, dst_p, z128)
    x1, y1 = _stage_first(g[0, :N], g[1, :N], ea0, ea1, We0, b0r, Wx1)
    g = _sc_agg(y1, src_p, dst_p, z128)
    x2, y2 = _stage_mid(g[0, :N], g[1, :N], ea0, ea1, x1, We1, b1r, Wx2)
    g = _sc_agg(y2, src_p, dst_p, z128)
    x3 = _stage_last(g[0, :N], g[1, :N], ea0, ea1, x2, We2, b2r)
    return x3
